# baseline (speedup 1.0000x reference)
import functools

import jax
import jax.numpy as jnp
from jax import lax
from jax.experimental import pallas as pl
from jax.experimental.pallas import tpu as pltpu
from jax.experimental.pallas import tpu_sc as plsc

N_ROWS = 8192
N_CODES = 8192
C_DIM = 32
ROW_BLK = 256
N_BLKS = N_ROWS // ROW_BLK

NC = 2
NS = 16
NW = NC * NS
B_PER_W = N_ROWS // NW
LANES = 16

_PREC = lax.Precision.DEFAULT


_SEG = (0, 2816, 5632, 8192)


def _argmin_body(x_ref, xsq_ref, cb_ref, cbsq_ref, idx_ref):
    dot = lax.dot_general(
        x_ref[...], cb_ref[...],
        dimension_numbers=(((1,), (1,)), ((), ())),
        precision=_PREC,
        preferred_element_type=jnp.float32,
    )
    d2 = (xsq_ref[...] + cbsq_ref[...]) - 2.0 * dot
    dist = jnp.sqrt(jnp.maximum(d2, 0.0))
    ids = lax.broadcasted_iota(jnp.int32, (ROW_BLK, N_CODES), 1)

    acc_v = None
    for s, e in zip(_SEG[:-1], _SEG[1:]):
        seg = dist[:, s:e]
        m = jnp.min(seg, axis=1, keepdims=True)
        cand = jnp.where(seg == m, ids[:, s:e], jnp.int32(2**31 - 1))
        i = jnp.min(cand, axis=1, keepdims=True)
        m_r = m.astype(jnp.bfloat16).astype(jnp.float32)
        if acc_v is None:
            acc_v, acc_i = m_r, i
        else:
            take = m < acc_v
            acc_i = jnp.where(take, i, acc_i)
            acc_v = jnp.where(take, m_r, acc_v)
    idx_ref[0, 0, :] = acc_i[:, 0]


def _tc_argmin(x, x_sq, codebook, cb_sq):
    out = pl.pallas_call(
        _argmin_body,
        grid=(N_BLKS,),
        in_specs=[
            pl.BlockSpec((ROW_BLK, C_DIM), lambda i: (i, 0)),
            pl.BlockSpec((ROW_BLK, 1), lambda i: (i, 0)),
            pl.BlockSpec((N_CODES, C_DIM), lambda i: (0, 0)),
            pl.BlockSpec((1, N_CODES), lambda i: (0, 0)),
        ],
        out_specs=pl.BlockSpec((1, 1, ROW_BLK), lambda i: (i, 0, 0)),
        out_shape=jax.ShapeDtypeStruct((N_BLKS, 1, ROW_BLK), jnp.int32),
    )(x, x_sq, codebook, cb_sq)
    return out.reshape(N_ROWS)


def _sc_gather_loss(codebook, idx, x):
    mesh = plsc.VectorSubcoreMesh(core_axis_name="c", subcore_axis_name="s")

    @functools.partial(
        pl.kernel,
        mesh=mesh,
        out_type=[
            jax.ShapeDtypeStruct((N_ROWS, C_DIM), jnp.float32),
            jax.ShapeDtypeStruct((NW, LANES), jnp.float32),
        ],
        scratch_types=[
            pltpu.VMEM((B_PER_W,), jnp.int32),
            pltpu.VMEM((B_PER_W, C_DIM), jnp.float32),
            pltpu.VMEM((B_PER_W, C_DIM), jnp.float32),
            pltpu.VMEM((LANES,), jnp.float32),
            pltpu.SemaphoreType.DMA,
        ],
        compiler_params=pltpu.CompilerParams(use_tc_tiling_on_sc=False),
    )
    def k(cb_hbm, idx_hbm, x_hbm, out_hbm, loss_hbm, idx_v, rows_v, x_v, loss_v, sem):
        wid = lax.axis_index("s") * NC + lax.axis_index("c")
        base = wid * B_PER_W
        pltpu.sync_copy(idx_hbm.at[pl.ds(base, B_PER_W)], idx_v)
        pltpu.async_copy(cb_hbm.at[idx_v], rows_v, sem).wait()
        pltpu.sync_copy(rows_v, out_hbm.at[pl.ds(base, B_PER_W)])
        pltpu.sync_copy(x_hbm.at[pl.ds(base, B_PER_W)], x_v)

        def body(i, acc):
            a0, a1 = acc
            d0 = rows_v[i, pl.ds(0, LANES)] - x_v[i, pl.ds(0, LANES)]
            d1 = rows_v[i, pl.ds(LANES, LANES)] - x_v[i, pl.ds(LANES, LANES)]
            return (a0 + d0 * d0, a1 + d1 * d1)

        z16 = jnp.zeros((LANES,), jnp.float32)
        a0, a1 = lax.fori_loop(0, B_PER_W, body, (z16, z16))
        loss_v[...] = a0 + a1
        pltpu.sync_copy(loss_v, loss_hbm.at[wid])

    return k(codebook, idx, x)


def kernel(z, codebook):
    B, C, H, W = z.shape
    x = jnp.transpose(z, (0, 2, 3, 1)).reshape(N_ROWS, C)
    x_sq = jnp.sum(x * x, axis=-1, keepdims=True)
    cb_sq = jnp.sum(codebook * codebook, axis=-1).reshape(1, N_CODES)

    idx = _tc_argmin(x, x_sq, codebook, cb_sq)
    q, loss_part = _sc_gather_loss(codebook, idx, x)

    loss = jnp.sum(loss_part) / (N_ROWS * C_DIM)
    z_out = jnp.transpose(q.reshape(B, H, W, C), (0, 3, 1, 2))
    return (z_out, loss, loss)
